# baseline (speedup 1.0000x reference)
import functools

import jax
import jax.numpy as jnp
from jax import lax
from jax.experimental import pallas as pl
from jax.experimental.pallas import tpu as pltpu
from jax.experimental.pallas import tpu_sc as plsc

SF = 2
S = SF ** 3
D = 64
N = 16384
IN_CH = 512
MID_CH = 512
C = MID_CH // S
OUT_CH = 64
NS = N * S
NQ = S // 2
PW = 2 * OUT_CH

M_TILE = 1024
NSPLIT = 4
NH = N // NSPLIT


def _mm_body(x_ref, wmid_ref, wout_ref, y_ref, wcomb_ref):
    @pl.when(pl.program_id(0) == 0)
    def _():
        wout = wout_ref[...]
        for l in range(S):
            wcomb_ref[:, l * C:(l + 1) * C] = jnp.dot(
                wmid_ref[:, l * C:(l + 1) * C], wout,
                preferred_element_type=jnp.float32)

    x = x_ref[...]
    for q in range(NQ):
        y_ref[q, :, :] = jnp.dot(x, wcomb_ref[:, q * PW:(q + 1) * PW],
                                 preferred_element_type=jnp.float32)


def _make_matmul(half):
    base = half * (NH // M_TILE)
    return pl.pallas_call(
        _mm_body,
        grid=(NH // M_TILE,),
        in_specs=[
            pl.BlockSpec((M_TILE, IN_CH), lambda m: (m + base, 0)),
            pl.BlockSpec((IN_CH, MID_CH), lambda m: (0, 0)),
            pl.BlockSpec((C, OUT_CH), lambda m: (0, 0)),
        ],
        out_specs=pl.BlockSpec((NQ, M_TILE, PW), lambda m: (0, m, 0)),
        out_shape=jax.ShapeDtypeStruct((NQ, NH, PW), jnp.float32),
        scratch_shapes=[pltpu.VMEM((IN_CH, MID_CH), jnp.float32)],
    )


_matmuls = [_make_matmul(h) for h in range(NSPLIT)]

_NC = 2
_NSUB = 16
_L = 16
_NW = _NC * _NSUB
P_PER_W = N // _NW
CHUNK = 32
_PAIRS = CHUNK * NQ
NCHUNK = P_PER_W // CHUNK


def _searchsorted(keys_ref, q):
    def step(_, lohi):
        lo, hi = lohi
        mid = (lo + hi) >> 1
        kv = plsc.load_gather(keys_ref, [mid])
        pred = kv < q
        return (jnp.where(pred, mid + 1, lo), jnp.where(pred, hi, mid))

    lo0 = jnp.zeros((_L,), jnp.int32)
    hi0 = jnp.full((_L,), N, jnp.int32)
    lo, _ = lax.fori_loop(0, 14, step, (lo0, hi0))
    return lo


_sc_mesh = plsc.VectorSubcoreMesh(core_axis_name="c", subcore_axis_name="s")


@functools.partial(
    pl.kernel,
    out_type=jax.ShapeDtypeStruct((NQ * N,), jnp.int32),
    mesh=_sc_mesh,
    scratch_types=[
        pltpu.VMEM((N,), jnp.int32),
        pltpu.VMEM((NQ * P_PER_W,), jnp.int32),
    ],
    compiler_params=pltpu.CompilerParams(needs_layout_passes=False),
)
def _sc_ranks(key_hbm, fwd_hbm, keys, fwdbuf):
    c = lax.axis_index("c")
    s = lax.axis_index("s")
    lanes = lax.iota(jnp.int32, _L)
    pltpu.sync_copy(key_hbm, keys)
    w0 = (s * _NC + c) * P_PER_W

    def vreg_body(v, _):
        kp = keys[pl.ds(w0 + v * _L, _L)]
        i = kp >> (2 * 6)
        j = (kp >> 6) & (D - 1)
        p = w0 + v * _L + lanes
        iv = i * (D * D)
        s1 = _searchsorted(keys, iv)
        s2 = _searchsorted(keys, iv + j * D)
        s3 = _searchsorted(keys, iv + D * D)
        s4 = _searchsorted(keys, iv + j * D + D)
        base = 4 * s1 + 2 * (s2 - s1) + (p - s2)
        c_di = 2 * (s3 - s1)
        c_dj = s4 - s2
        for q in range(NQ):
            r = base
            if q & 2:
                r = r + c_di
            if q & 1:
                r = r + c_dj
            fwdbuf[pl.ds(q * P_PER_W + v * _L, _L)] = r
        return 0

    lax.fori_loop(0, P_PER_W // _L, vreg_body, 0)
    for q in range(NQ):
        pltpu.sync_copy(fwdbuf.at[pl.ds(q * P_PER_W, P_PER_W)],
                        fwd_hbm.at[pl.ds(q * N + w0, P_PER_W)])


P_H = NH // _NW
NCHUNK_H = P_H // CHUNK


def _make_scatter(half):
    @functools.partial(
        pl.kernel,
        out_type=(),
        mesh=_sc_mesh,
        scratch_types=[
            pltpu.VMEM((_PAIRS, PW), jnp.float32),
            pltpu.VMEM((_PAIRS, PW), jnp.float32),
            pltpu.VMEM((_PAIRS, PW), jnp.float32),
            pltpu.VMEM((_PAIRS, PW), jnp.float32),
            pltpu.VMEM((_PAIRS,), jnp.int32),
            pltpu.VMEM((_PAIRS,), jnp.int32),
            pltpu.VMEM((_PAIRS,), jnp.int32),
            pltpu.VMEM((_PAIRS,), jnp.int32),
            pltpu.VMEM((_PAIRS,), jnp.int32),
            pltpu.VMEM((_PAIRS,), jnp.int32),
            pltpu.SemaphoreType.DMA,
            pltpu.SemaphoreType.DMA,
            pltpu.SemaphoreType.DMA,
            pltpu.SemaphoreType.DMA,
        ],
        compiler_params=pltpu.CompilerParams(needs_layout_passes=False,
                                             use_tc_tiling_on_sc=False),
    )
    def _sc_scatter(fwd_hbm, y_hbm, out_hbm, re0, re1, ro0, ro1, fw0, fw1,
                    ie0, ie1, io0, io1, semi0, semi1, semo0, semo1):
        c = lax.axis_index("c")
        s = lax.axis_index("s")
        w0 = (s * _NC + c) * P_H
        g0 = half * NH + w0
        bufs = ((re0, ro0, fw0, ie0, io0, semi0, semo0),
                (re1, ro1, fw1, ie1, io1, semi1, semo1))

        def start_in(t, re, ro, fwb, semi):
            pl0 = w0 + t * CHUNK
            pg0 = g0 + t * CHUNK
            ds = []
            for q in range(NQ):
                ds.append(pltpu.async_copy(
                    y_hbm.at[q, pl.ds(pl0, CHUNK), pl.ds(0, OUT_CH)],
                    re.at[pl.ds(q * CHUNK, CHUNK), pl.ds(0, OUT_CH)], semi))
                ds.append(pltpu.async_copy(
                    y_hbm.at[q, pl.ds(pl0, CHUNK), pl.ds(OUT_CH, OUT_CH)],
                    ro.at[pl.ds(q * CHUNK, CHUNK), pl.ds(0, OUT_CH)], semi))
                ds.append(pltpu.async_copy(
                    fwd_hbm.at[pl.ds(q * N + pg0, CHUNK)],
                    fwb.at[pl.ds(q * CHUNK, CHUNK)], semi))
            return ds

        def do_chunk(t, buf, first):
            re, ro, fwb, ie, io, semi, semo = buf
            if not first:
                pltpu.make_async_copy(re, out_hbm.at[ie], semo).wait()
                pltpu.make_async_copy(ro, out_hbm.at[io], semo).wait()
            descs = start_in(t, re, ro, fwb, semi)
            for d in descs:
                d.wait()
            for v in range(_PAIRS // _L):
                f = fwb[pl.ds(v * _L, _L)]
                ie[pl.ds(v * _L, _L)] = 2 * f
                io[pl.ds(v * _L, _L)] = 2 * f + 1
            pltpu.async_copy(re, out_hbm.at[ie], semo)
            pltpu.async_copy(ro, out_hbm.at[io], semo)

        do_chunk(0, bufs[0], True)
        do_chunk(1, bufs[1], True)

        def body(u, _):
            do_chunk(2 * u, bufs[0], False)
            do_chunk(2 * u + 1, bufs[1], False)
            return 0

        lax.fori_loop(1, NCHUNK_H // 2, body, 0)
        for re, ro, fwb, ie, io, semi, semo in bufs:
            pltpu.make_async_copy(re, out_hbm.at[ie], semo).wait()
            pltpu.make_async_copy(ro, out_hbm.at[io], semo).wait()

    return _sc_scatter


_scatters = [_make_scatter(h) for h in range(NSPLIT)]


def kernel(x, ijk, W_mid, W_out):
    ijk32 = ijk.astype(jnp.int32)
    key = ijk32[:, 0] * (D * D) + ijk32[:, 1] * D + ijk32[:, 2]
    fwd = _sc_ranks(key)
    out_ref = jax.new_ref(lax.empty((NS, PW), jnp.float32))
    for h in range(NSPLIT):
        y_h = _matmuls[h](x, W_mid, W_out)
        _scatters[h](fwd, y_h, out_ref)
    out = out_ref[...]
    return out[:, :OUT_CH]


# speedup vs baseline: 1.0973x; 1.0973x over previous
import functools

import jax
import jax.numpy as jnp
from jax import lax
from jax.experimental import pallas as pl
from jax.experimental.pallas import tpu as pltpu
from jax.experimental.pallas import tpu_sc as plsc

SF = 2
S = SF ** 3
D = 64
N = 16384
IN_CH = 512
MID_CH = 512
C = MID_CH // S
OUT_CH = 64
NS = N * S
NQ = S // 2
PW = 2 * OUT_CH

M_TILE = 2048
NSPLIT = 2
NH = N // NSPLIT


def _mm_body(x_ref, wmid_ref, wout_ref, y_ref, wcomb_ref):
    @pl.when(pl.program_id(0) == 0)
    def _():
        wout = wout_ref[...]
        for l in range(S):
            wcomb_ref[:, l * C:(l + 1) * C] = jnp.dot(
                wmid_ref[:, l * C:(l + 1) * C], wout,
                preferred_element_type=jnp.float32)

    x = x_ref[...]
    for q in range(NQ):
        y_ref[q, :, :] = jnp.dot(x, wcomb_ref[:, q * PW:(q + 1) * PW],
                                 preferred_element_type=jnp.float32)


def _make_matmul(half):
    base = half * (NH // M_TILE)
    return pl.pallas_call(
        _mm_body,
        grid=(NH // M_TILE,),
        in_specs=[
            pl.BlockSpec((M_TILE, IN_CH), lambda m: (m + base, 0)),
            pl.BlockSpec((IN_CH, MID_CH), lambda m: (0, 0)),
            pl.BlockSpec((C, OUT_CH), lambda m: (0, 0)),
        ],
        out_specs=pl.BlockSpec((NQ, M_TILE, PW), lambda m: (0, m, 0)),
        out_shape=jax.ShapeDtypeStruct((NQ, NH, PW), jnp.float32),
        scratch_shapes=[pltpu.VMEM((IN_CH, MID_CH), jnp.float32)],
    )


_matmuls = [_make_matmul(h) for h in range(NSPLIT)]

_NC = 2
_NSUB = 16
_L = 16
_NW = _NC * _NSUB
P_PER_W = N // _NW
CHUNK = 32
_PAIRS = CHUNK * NQ
NCHUNK = P_PER_W // CHUNK


def _searchsorted(keys_ref, q):
    def step(_, lohi):
        lo, hi = lohi
        mid = (lo + hi) >> 1
        kv = plsc.load_gather(keys_ref, [mid])
        pred = kv < q
        return (jnp.where(pred, mid + 1, lo), jnp.where(pred, hi, mid))

    lo0 = jnp.zeros((_L,), jnp.int32)
    hi0 = jnp.full((_L,), N, jnp.int32)
    lo, _ = lax.fori_loop(0, 14, step, (lo0, hi0))
    return lo


_sc_mesh = plsc.VectorSubcoreMesh(core_axis_name="c", subcore_axis_name="s")


@functools.partial(
    pl.kernel,
    out_type=jax.ShapeDtypeStruct((NQ * N,), jnp.int32),
    mesh=_sc_mesh,
    scratch_types=[
        pltpu.VMEM((N,), jnp.int32),
        pltpu.VMEM((NQ * P_PER_W,), jnp.int32),
    ],
    compiler_params=pltpu.CompilerParams(needs_layout_passes=False),
)
def _sc_ranks(key_hbm, fwd_hbm, keys, fwdbuf):
    c = lax.axis_index("c")
    s = lax.axis_index("s")
    lanes = lax.iota(jnp.int32, _L)
    pltpu.sync_copy(key_hbm, keys)
    w0 = (s * _NC + c) * P_PER_W

    def vreg_body(v, _):
        kp = keys[pl.ds(w0 + v * _L, _L)]
        i = kp >> (2 * 6)
        j = (kp >> 6) & (D - 1)
        p = w0 + v * _L + lanes
        iv = i * (D * D)
        s1 = _searchsorted(keys, iv)
        s2 = _searchsorted(keys, iv + j * D)
        s3 = _searchsorted(keys, iv + D * D)
        s4 = _searchsorted(keys, iv + j * D + D)
        base = 4 * s1 + 2 * (s2 - s1) + (p - s2)
        c_di = 2 * (s3 - s1)
        c_dj = s4 - s2
        for q in range(NQ):
            r = base
            if q & 2:
                r = r + c_di
            if q & 1:
                r = r + c_dj
            fwdbuf[pl.ds(q * P_PER_W + v * _L, _L)] = r
        return 0

    lax.fori_loop(0, P_PER_W // _L, vreg_body, 0)
    for q in range(NQ):
        pltpu.sync_copy(fwdbuf.at[pl.ds(q * P_PER_W, P_PER_W)],
                        fwd_hbm.at[pl.ds(q * N + w0, P_PER_W)])


P_H = NH // _NW
NCHUNK_H = P_H // CHUNK


def _make_scatter(half):
    @functools.partial(
        pl.kernel,
        out_type=(),
        mesh=_sc_mesh,
        scratch_types=[
            pltpu.VMEM((_PAIRS, PW), jnp.float32),
            pltpu.VMEM((_PAIRS, PW), jnp.float32),
            pltpu.VMEM((_PAIRS, PW), jnp.float32),
            pltpu.VMEM((_PAIRS, PW), jnp.float32),
            pltpu.VMEM((_PAIRS,), jnp.int32),
            pltpu.VMEM((_PAIRS,), jnp.int32),
            pltpu.VMEM((_PAIRS,), jnp.int32),
            pltpu.VMEM((_PAIRS,), jnp.int32),
            pltpu.VMEM((_PAIRS,), jnp.int32),
            pltpu.VMEM((_PAIRS,), jnp.int32),
            pltpu.SemaphoreType.DMA,
            pltpu.SemaphoreType.DMA,
            pltpu.SemaphoreType.DMA,
            pltpu.SemaphoreType.DMA,
        ],
        compiler_params=pltpu.CompilerParams(needs_layout_passes=False,
                                             use_tc_tiling_on_sc=False),
    )
    def _sc_scatter(fwd_hbm, y_hbm, out_hbm, re0, re1, ro0, ro1, fw0, fw1,
                    ie0, ie1, io0, io1, semi0, semi1, semo0, semo1):
        c = lax.axis_index("c")
        s = lax.axis_index("s")
        w0 = (s * _NC + c) * P_H
        g0 = half * NH + w0
        bufs = ((re0, ro0, fw0, ie0, io0, semi0, semo0),
                (re1, ro1, fw1, ie1, io1, semi1, semo1))

        def start_in(t, re, ro, fwb, semi):
            pl0 = w0 + t * CHUNK
            pg0 = g0 + t * CHUNK
            ds = []
            for q in range(NQ):
                ds.append(pltpu.async_copy(
                    y_hbm.at[q, pl.ds(pl0, CHUNK), pl.ds(0, OUT_CH)],
                    re.at[pl.ds(q * CHUNK, CHUNK), pl.ds(0, OUT_CH)], semi))
                ds.append(pltpu.async_copy(
                    y_hbm.at[q, pl.ds(pl0, CHUNK), pl.ds(OUT_CH, OUT_CH)],
                    ro.at[pl.ds(q * CHUNK, CHUNK), pl.ds(0, OUT_CH)], semi))
                ds.append(pltpu.async_copy(
                    fwd_hbm.at[pl.ds(q * N + pg0, CHUNK)],
                    fwb.at[pl.ds(q * CHUNK, CHUNK)], semi))
            return ds

        def do_chunk(t, buf, first):
            re, ro, fwb, ie, io, semi, semo = buf
            if not first:
                pltpu.make_async_copy(re, out_hbm.at[ie], semo).wait()
                pltpu.make_async_copy(ro, out_hbm.at[io], semo).wait()
            descs = start_in(t, re, ro, fwb, semi)
            for d in descs:
                d.wait()
            for v in range(_PAIRS // _L):
                f = fwb[pl.ds(v * _L, _L)]
                ie[pl.ds(v * _L, _L)] = 2 * f
                io[pl.ds(v * _L, _L)] = 2 * f + 1
            pltpu.async_copy(re, out_hbm.at[ie], semo)
            pltpu.async_copy(ro, out_hbm.at[io], semo)

        do_chunk(0, bufs[0], True)
        do_chunk(1, bufs[1], True)

        def body(u, _):
            do_chunk(2 * u, bufs[0], False)
            do_chunk(2 * u + 1, bufs[1], False)
            return 0

        lax.fori_loop(1, NCHUNK_H // 2, body, 0)
        for re, ro, fwb, ie, io, semi, semo in bufs:
            pltpu.make_async_copy(re, out_hbm.at[ie], semo).wait()
            pltpu.make_async_copy(ro, out_hbm.at[io], semo).wait()

    return _sc_scatter


_scatters = [_make_scatter(h) for h in range(NSPLIT)]


def kernel(x, ijk, W_mid, W_out):
    ijk32 = ijk.astype(jnp.int32)
    key = ijk32[:, 0] * (D * D) + ijk32[:, 1] * D + ijk32[:, 2]
    fwd = _sc_ranks(key)
    out_ref = jax.new_ref(lax.empty((NS, PW), jnp.float32))
    for h in range(NSPLIT):
        y_h = _matmuls[h](x, W_mid, W_out)
        _scatters[h](fwd, y_h, out_ref)
    out = out_ref[...]
    return out[:, :OUT_CH]
